# baseline (speedup 1.0000x reference)
import jax
import jax.numpy as jnp
from jax.experimental import pallas as pl

M = 16384
N = 16384
BM = 256


BN = 8192
BMW = 512


def _mv_body(a_ref, x_ref, o_ref):
    j = pl.program_id(1)
    part = jnp.sum(a_ref[...] * x_ref[...], axis=1)

    @pl.when(j == 0)
    def _():
        o_ref[...] = part

    @pl.when(j != 0)
    def _():
        o_ref[...] += part


def kernel(A, x):
    return pl.pallas_call(
        _mv_body,
        grid=(M // BMW, N // BN),
        in_specs=[
            pl.BlockSpec((BMW, BN), lambda i, j: (i, j)),
            pl.BlockSpec((1, BN), lambda i, j: (0, j)),
        ],
        out_specs=pl.BlockSpec((BMW,), lambda i, j: (i,)),
        out_shape=jax.ShapeDtypeStruct((M,), jnp.float32),
    )(A, x.reshape(1, N))


# speedup vs baseline: 1.0017x; 1.0017x over previous
import jax
import jax.numpy as jnp
from jax.experimental import pallas as pl

M = 16384
N = 16384
BM = 256


def _mv_body(a_ref, x_ref, o_ref):
    o_ref[...] = jnp.sum(a_ref[...] * x_ref[...], axis=1)


def kernel(A, x):
    return pl.pallas_call(
        _mv_body,
        grid=(M // BM,),
        in_specs=[
            pl.BlockSpec((BM, N), lambda i: (i, 0)),
            pl.BlockSpec((1, N), lambda i: (0, 0)),
        ],
        out_specs=pl.BlockSpec((BM,), lambda i: (i,)),
        out_shape=jax.ShapeDtypeStruct((M,), jnp.float32),
    )(A, x.reshape(1, N))
